# baseline (speedup 1.0000x reference)
import functools

import jax
import jax.numpy as jnp
from jax.experimental import pallas as pl
from jax.experimental.pallas import tpu as pltpu

_CHUNKS = 4


def _scse_pipe_body(x_hbm, w1t_ref, b1t_ref, w2t_ref, b2t_ref, wst_ref,
                    bs_ref, o_hbm, in_buf, out_buf, rsem, wsem,
                    *, n_planes, n_ch, inv_hw):
    cc = in_buf.shape[1] // n_ch

    def start_reads(n, s):
        for c in range(n_ch):
            pltpu.make_async_copy(
                x_hbm.at[n, pl.ds(c * cc, cc), :],
                in_buf.at[s, pl.ds(c * cc, cc), :],
                rsem.at[s, c],
            ).start()

    def wait_reads(s):
        for c in range(n_ch):
            pltpu.make_async_copy(
                in_buf.at[s, pl.ds(c * cc, cc), :],
                in_buf.at[s, pl.ds(c * cc, cc), :],
                rsem.at[s, c],
            ).wait()

    def start_writes(n, s):
        for c in range(n_ch):
            pltpu.make_async_copy(
                out_buf.at[s, pl.ds(c * cc, cc), :],
                o_hbm.at[n, pl.ds(c * cc, cc), :],
                wsem.at[s, c],
            ).start()

    def wait_writes(s):
        for c in range(n_ch):
            pltpu.make_async_copy(
                out_buf.at[s, pl.ds(c * cc, cc), :],
                out_buf.at[s, pl.ds(c * cc, cc), :],
                wsem.at[s, c],
            ).wait()

    start_reads(0, 0)

    def body(n, carry):
        s = jax.lax.rem(n, 2)
        ns = jax.lax.rem(n + 1, 2)

        @pl.when(n + 1 < n_planes)
        def _():
            start_reads(n + 1, ns)

        wait_reads(s)
        xv = in_buf[s]

        pooled = jnp.sum(xv, axis=1, keepdims=True,
                         dtype=jnp.float32) * inv_hw
        h = jnp.maximum(
            jnp.dot(w1t_ref[...], pooled,
                    preferred_element_type=jnp.float32) + b1t_ref[...],
            0.0,
        )
        cgate = jax.nn.sigmoid(
            jnp.dot(w2t_ref[...], h,
                    preferred_element_type=jnp.float32) + b2t_ref[...]
        )

        slogit = jnp.dot(wst_ref[...], xv,
                         preferred_element_type=jnp.float32) + bs_ref[0, 0]
        sgate = jax.nn.sigmoid(slogit)

        @pl.when(n >= 2)
        def _():
            wait_writes(s)

        out_buf[s] = xv * (cgate + sgate)
        start_writes(n, s)
        return carry

    jax.lax.fori_loop(0, n_planes, body, 0)
    wait_writes(jax.lax.rem(jnp.int32(n_planes - 2), 2))
    wait_writes(jax.lax.rem(jnp.int32(n_planes - 1), 2))


def kernel(x_nchw, w1, b1, w2, b2, ws, bs):
    N, C, H, W = x_nchw.shape
    HW = H * W
    Cr = w1.shape[1]

    x = x_nchw.reshape(N, C, HW)
    w1t = w1.T
    b1t = b1.reshape(Cr, 1)
    w2t = w2.T
    b2t = b2.reshape(C, 1)
    wst = ws.reshape(1, C)
    bs2 = bs.reshape(1, 1)

    body = functools.partial(
        _scse_pipe_body, n_planes=N, n_ch=_CHUNKS, inv_hw=1.0 / float(HW))

    out = pl.pallas_call(
        body,
        out_shape=jax.ShapeDtypeStruct((N, C, HW), x.dtype),
        in_specs=[
            pl.BlockSpec(memory_space=pl.ANY),
            pl.BlockSpec(memory_space=pltpu.MemorySpace.VMEM),
            pl.BlockSpec(memory_space=pltpu.MemorySpace.VMEM),
            pl.BlockSpec(memory_space=pltpu.MemorySpace.VMEM),
            pl.BlockSpec(memory_space=pltpu.MemorySpace.VMEM),
            pl.BlockSpec(memory_space=pltpu.MemorySpace.VMEM),
            pl.BlockSpec(memory_space=pltpu.MemorySpace.VMEM),
        ],
        out_specs=pl.BlockSpec(memory_space=pl.ANY),
        scratch_shapes=[
            pltpu.VMEM((2, C, HW), jnp.float32),
            pltpu.VMEM((2, C, HW), jnp.float32),
            pltpu.SemaphoreType.DMA((2, _CHUNKS)),
            pltpu.SemaphoreType.DMA((2, _CHUNKS)),
        ],
        compiler_params=pltpu.CompilerParams(
            vmem_limit_bytes=56 * 1024 * 1024,
        ),
    )(x, w1t, b1t, w2t, b2t, wst, bs2)
    return out.reshape(N, C, H, W)


# speedup vs baseline: 1.0020x; 1.0020x over previous
import functools

import jax
import jax.numpy as jnp
from jax.experimental import pallas as pl
from jax.experimental.pallas import tpu as pltpu

_CHUNKS = 4


def _scse_pipe_body(x_hbm, w1t_ref, b1t_ref, w2t_ref, b2t_ref, wst_ref,
                    bs_ref, o_hbm, in_buf, out_buf, rsem, wsem,
                    *, n_planes, n_ch, inv_hw):
    cc = in_buf.shape[1] // n_ch

    def start_reads(n, s):
        for c in range(n_ch):
            pltpu.make_async_copy(
                x_hbm.at[n, pl.ds(c * cc, cc), :],
                in_buf.at[s, pl.ds(c * cc, cc), :],
                rsem.at[s, c],
            ).start()

    def wait_reads(s):
        for c in range(n_ch):
            pltpu.make_async_copy(
                in_buf.at[s, pl.ds(c * cc, cc), :],
                in_buf.at[s, pl.ds(c * cc, cc), :],
                rsem.at[s, c],
            ).wait()

    def start_writes(n, s):
        for c in range(n_ch):
            pltpu.make_async_copy(
                out_buf.at[s, pl.ds(c * cc, cc), :],
                o_hbm.at[n, pl.ds(c * cc, cc), :],
                wsem.at[s, c],
            ).start(priority=1)

    def wait_writes(s):
        for c in range(n_ch):
            pltpu.make_async_copy(
                out_buf.at[s, pl.ds(c * cc, cc), :],
                out_buf.at[s, pl.ds(c * cc, cc), :],
                wsem.at[s, c],
            ).wait()

    start_reads(0, 0)

    def body(n, carry):
        s = jax.lax.rem(n, 2)
        ns = jax.lax.rem(n + 1, 2)

        @pl.when(n + 1 < n_planes)
        def _():
            start_reads(n + 1, ns)

        wait_reads(s)
        xv = in_buf[s]

        pooled = jnp.sum(xv, axis=1, keepdims=True,
                         dtype=jnp.float32) * inv_hw
        h = jnp.maximum(
            jnp.dot(w1t_ref[...], pooled,
                    preferred_element_type=jnp.float32) + b1t_ref[...],
            0.0,
        )
        cgate = jax.nn.sigmoid(
            jnp.dot(w2t_ref[...], h,
                    preferred_element_type=jnp.float32) + b2t_ref[...]
        )

        slogit = jnp.dot(wst_ref[...], xv,
                         preferred_element_type=jnp.float32) + bs_ref[0, 0]
        sgate = jax.nn.sigmoid(slogit)

        @pl.when(n >= 2)
        def _():
            wait_writes(s)

        out_buf[s] = xv * (cgate + sgate)
        start_writes(n, s)
        return carry

    jax.lax.fori_loop(0, n_planes, body, 0)
    wait_writes(jax.lax.rem(jnp.int32(n_planes - 2), 2))
    wait_writes(jax.lax.rem(jnp.int32(n_planes - 1), 2))


def kernel(x_nchw, w1, b1, w2, b2, ws, bs):
    N, C, H, W = x_nchw.shape
    HW = H * W
    Cr = w1.shape[1]

    x = x_nchw.reshape(N, C, HW)
    w1t = w1.T
    b1t = b1.reshape(Cr, 1)
    w2t = w2.T
    b2t = b2.reshape(C, 1)
    wst = ws.reshape(1, C)
    bs2 = bs.reshape(1, 1)

    body = functools.partial(
        _scse_pipe_body, n_planes=N, n_ch=_CHUNKS, inv_hw=1.0 / float(HW))

    out = pl.pallas_call(
        body,
        out_shape=jax.ShapeDtypeStruct((N, C, HW), x.dtype),
        in_specs=[
            pl.BlockSpec(memory_space=pl.ANY),
            pl.BlockSpec(memory_space=pltpu.MemorySpace.VMEM),
            pl.BlockSpec(memory_space=pltpu.MemorySpace.VMEM),
            pl.BlockSpec(memory_space=pltpu.MemorySpace.VMEM),
            pl.BlockSpec(memory_space=pltpu.MemorySpace.VMEM),
            pl.BlockSpec(memory_space=pltpu.MemorySpace.VMEM),
            pl.BlockSpec(memory_space=pltpu.MemorySpace.VMEM),
        ],
        out_specs=pl.BlockSpec(memory_space=pl.ANY),
        scratch_shapes=[
            pltpu.VMEM((2, C, HW), jnp.float32),
            pltpu.VMEM((2, C, HW), jnp.float32),
            pltpu.SemaphoreType.DMA((2, _CHUNKS)),
            pltpu.SemaphoreType.DMA((2, _CHUNKS)),
        ],
        compiler_params=pltpu.CompilerParams(
            vmem_limit_bytes=56 * 1024 * 1024,
        ),
    )(x, w1t, b1t, w2t, b2t, wst, bs2)
    return out.reshape(N, C, H, W)


# speedup vs baseline: 1.0244x; 1.0224x over previous
import functools

import jax
import jax.numpy as jnp
from jax.experimental import pallas as pl
from jax.experimental.pallas import tpu as pltpu


def _scse_plane_kernel(x_ref, w1t_ref, b1t_ref, w2t_ref, b2t_ref, wst_ref,
                       bs_ref, o_ref, *, inv_hw):
    for i in range(x_ref.shape[0]):
        xv = x_ref[i]

        pooled = jnp.sum(xv, axis=1, keepdims=True,
                         dtype=jnp.float32) * inv_hw
        h = jnp.maximum(
            jnp.dot(w1t_ref[...], pooled,
                    preferred_element_type=jnp.float32) + b1t_ref[...],
            0.0,
        )
        cgate = jax.nn.sigmoid(
            jnp.dot(w2t_ref[...], h,
                    preferred_element_type=jnp.float32) + b2t_ref[...]
        )

        slogit = jnp.dot(wst_ref[...], xv,
                         preferred_element_type=jnp.float32) + bs_ref[0, 0]
        sgate = jax.nn.sigmoid(slogit)

        o_ref[i] = xv * (cgate + sgate)


def kernel(x_nchw, w1, b1, w2, b2, ws, bs):
    N, C, H, W = x_nchw.shape
    HW = H * W
    Cr = w1.shape[1]
    BB = 2

    x = x_nchw.reshape(N, C, HW)
    w1t = w1.T
    b1t = b1.reshape(Cr, 1)
    w2t = w2.T
    b2t = b2.reshape(C, 1)
    wst = ws.reshape(1, C)
    bs2 = bs.reshape(1, 1)

    body = functools.partial(_scse_plane_kernel, inv_hw=1.0 / float(HW))

    out = pl.pallas_call(
        body,
        out_shape=jax.ShapeDtypeStruct((N, C, HW), x.dtype),
        grid=(N // BB,),
        in_specs=[
            pl.BlockSpec((BB, C, HW), lambda n: (n, 0, 0)),
            pl.BlockSpec((Cr, C), lambda n: (0, 0)),
            pl.BlockSpec((Cr, 1), lambda n: (0, 0)),
            pl.BlockSpec((C, Cr), lambda n: (0, 0)),
            pl.BlockSpec((C, 1), lambda n: (0, 0)),
            pl.BlockSpec((1, C), lambda n: (0, 0)),
            pl.BlockSpec((1, 1), lambda n: (0, 0)),
        ],
        out_specs=pl.BlockSpec((BB, C, HW), lambda n: (n, 0, 0)),
        compiler_params=pltpu.CompilerParams(
            dimension_semantics=("parallel",),
            vmem_limit_bytes=56 * 1024 * 1024,
        ),
    )(x, w1t, b1t, w2t, b2t, wst, bs2)
    return out.reshape(N, C, H, W)
